# baseline (speedup 1.0000x reference)
import functools

import jax
import jax.numpy as jnp
from jax import lax
from jax.experimental import pallas as pl
from jax.experimental.pallas import tpu as pltpu
from jax.experimental.pallas import tpu_sc as plsc

N_ATOMS = 10000
F = 128
R = 64
N_PAIRS = 320000
NC = 2
NS = 16
NW = NC * NS
PAIRS_PER_TILE = N_PAIRS // NW
CHUNK = 200
SUB = 40
NSUB = CHUNK // SUB
ITERS = PAIRS_PER_TILE // CHUNK
NBUF = 2
N_PAD = 10240
ROWS_PER_TILE = N_PAD // NS
ZH = 80


def _sc_scatter_body(radial_hbm, idx0_hbm, idx1_hbm, out_hbm,
                     idx_v, rows_v, sems, acc):
    c = lax.axis_index("c")
    s = lax.axis_index("s")
    wid = c * NS + s

    z16 = jnp.zeros((16,), jnp.float32)
    for i in range(ZH):
        for k in range(R // 16):
            rows_v[0, i, pl.ds(k * 16, 16)] = z16
    row0 = s * ROWS_PER_TILE
    for j in range(ROWS_PER_TILE // ZH):
        pltpu.sync_copy(rows_v.at[0, pl.ds(0, ZH)], acc.at[pl.ds(row0 + j * ZH, ZH)])
    plsc.subcore_barrier()

    pair0 = wid * PAIRS_PER_TILE

    def issue_load(j, b):
        pb = pair0 + j * CHUNK
        pltpu.async_copy(idx0_hbm.at[pl.ds(pb, CHUNK)], idx_v.at[b, 0], sems.at[b])
        pltpu.async_copy(idx1_hbm.at[pl.ds(pb, CHUNK)], idx_v.at[b, 1], sems.at[b])
        pltpu.async_copy(radial_hbm.at[pl.ds(pb, CHUNK)], rows_v.at[b], sems.at[b])

    def wait_load(j, b):
        pb = pair0 + j * CHUNK
        pltpu.make_async_copy(idx0_hbm.at[pl.ds(pb, CHUNK)], idx_v.at[b, 0], sems.at[b]).wait()
        pltpu.make_async_copy(idx1_hbm.at[pl.ds(pb, CHUNK)], idx_v.at[b, 1], sems.at[b]).wait()
        pltpu.make_async_copy(radial_hbm.at[pl.ds(pb, CHUNK)], rows_v.at[b], sems.at[b]).wait()

    for b in range(NBUF):
        issue_load(b, b)

    @pl.loop(0, ITERS, step=NBUF)
    def _pipeline(g):
        for b in range(NBUF):
            j = g + b
            wait_load(j, b)
            for k in range(NSUB):
                src = rows_v.at[b, pl.ds(k * SUB, SUB)]
                pltpu.sync_copy(src, acc.at[idx_v.at[b, 0, pl.ds(k * SUB, SUB)]], add=True)
                pltpu.sync_copy(src, acc.at[idx_v.at[b, 1, pl.ds(k * SUB, SUB)]], add=True)
            nj = j + NBUF

            @pl.when(nj < ITERS)
            def _():
                issue_load(nj, b)

    plsc.subcore_barrier()

    for j in range(ROWS_PER_TILE // ZH):
        pltpu.sync_copy(acc.at[pl.ds(row0 + j * ZH, ZH)], rows_v.at[0, pl.ds(0, ZH)])
        pltpu.sync_copy(rows_v.at[0, pl.ds(0, ZH)], out_hbm.at[c, pl.ds(row0 + j * ZH, ZH)])


@jax.jit
def _sc_scatter(radial_aev, idx0, idx1):
    mesh = plsc.VectorSubcoreMesh(core_axis_name="c", subcore_axis_name="s")
    return pl.kernel(
        _sc_scatter_body,
        out_type=jax.ShapeDtypeStruct((NC, N_PAD, R), jnp.float32),
        mesh=mesh,
        compiler_params=pltpu.CompilerParams(use_tc_tiling_on_sc=False),
        scratch_types=[
            pltpu.VMEM((NBUF, 2, CHUNK), jnp.int32),
            pltpu.VMEM((NBUF, CHUNK, R), jnp.float32),
            pltpu.SemaphoreType.DMA((NBUF,)),
            pltpu.VMEM_SHARED((N_PAD, R), jnp.float32),
        ],
    )(radial_aev, idx0, idx1)


_LOG2 = 0.6931471805599453


def _sp(x):
    return jnp.maximum(x, 0.0) + jnp.log(1.0 + jnp.exp(-jnp.abs(x))) - _LOG2


def _mm(x, w):
    return lax.dot_general(x, w, (((1,), (1,)), ((), ())),
                           preferred_element_type=jnp.float32)


def _dense_body(species_ref, feat_ref, part_ref, *rest):
    wrefs = rest[:-2]
    energy_ref, outfeat_ref = rest[-2:]
    w = [r[...] for r in wrefs]
    (WI, bI, WJ, bJ, Wg,
     i10, i11, i12, i13, i20, i21, i22, i23, i30, i31, i32, i33,
     Wint, bint, gate,
     a10, a11, a12, a13, a20, a21, a22, a23,
     o10, o11, o12, o13,
     Wout, bout) = w

    def res_block(x, W1, b1, W2, b2):
        out = _mm(_sp(x), W1) + b1
        return _mm(_sp(out), W2) + b2 + x

    x = feat_ref[...]
    mask = species_ref[...] != -1
    af = _sp(x)
    g = _sp(_mm(af, WJ) + bJ)
    protoI = _sp(_mm(af, WI) + bI)
    A = part_ref[0] + part_ref[1]
    S = _mm(A, Wg)
    proto = S * g + jnp.where(mask, protoI, 0.0)
    msg = res_block(proto, i10, i11, i12, i13)
    msg = res_block(msg, i20, i21, i22, i23)
    msg = res_block(msg, i30, i31, i32, i33)
    dense = x * gate + _mm(_sp(msg), Wint) + bint
    dense = res_block(dense, a10, a11, a12, a13)
    dense = res_block(dense, a20, a21, a22, a23)
    t = res_block(dense, o10, o11, o12, o13)
    e = jnp.sum(_sp(t) * Wout + bout, axis=1, keepdims=True)
    energy_ref[...] = jnp.where(mask, e, 0.0)
    outfeat_ref[...] = jnp.where(mask, dense, 0.0)


def _dense_chain(species_flat, features, partial, weights, block_rows=2000):
    grid = (N_ATOMS // block_rows,)
    row_spec = lambda cols: pl.BlockSpec((block_rows, cols), lambda i: (i, 0))
    in_specs = [
        row_spec(1),
        row_spec(F),
        pl.BlockSpec((NC, block_rows, R), lambda i: (0, i, 0)),
    ] + [pl.BlockSpec(wi.shape, lambda i: (0, 0)) for wi in weights]
    out_specs = [row_spec(1), row_spec(F)]
    energies, out_features = pl.pallas_call(
        _dense_body,
        grid=grid,
        in_specs=in_specs,
        out_specs=out_specs,
        out_shape=[
            jax.ShapeDtypeStruct((N_ATOMS, 1), jnp.float32),
            jax.ShapeDtypeStruct((N_ATOMS, F), jnp.float32),
        ],
    )(species_flat, features, partial, *weights)
    return energies, out_features


def kernel(species, features, radial_aev, atom_index12, params):
    idx = atom_index12.astype(jnp.int32)
    partial = _sc_scatter(radial_aev, idx[0], idx[1])

    def lin2(p):
        W, b = p
        return [W, b.reshape(1, F)]

    pr = params
    weights = (
        lin2(pr['linearI']) + lin2(pr['linearJ']) + [pr['gating_linear_W']]
        + [t for blk in pr['inter_res'] for p in blk for t in lin2(p)]
        + lin2(pr['interaction_linear'])
        + [pr['gating_vector'].reshape(1, F)]
        + [t for blk in pr['atomic_res'] for p in blk for t in lin2(p)]
        + [t for blk in pr['output_res'] for p in blk for t in lin2(p)]
        + [pr['output_linear'][0],
           jnp.broadcast_to(pr['output_linear'][1].reshape(1, 1) / F, (1, F))]
    )
    species_flat = species.reshape(-1, 1).astype(jnp.int32)
    energies, out_features = _dense_chain(species_flat, features, partial, weights)
    return energies.reshape(species.shape[0], species.shape[1]), out_features


# speedup vs baseline: 1.0014x; 1.0014x over previous
import functools

import jax
import jax.numpy as jnp
from jax import lax
from jax.experimental import pallas as pl
from jax.experimental.pallas import tpu as pltpu
from jax.experimental.pallas import tpu_sc as plsc

N_ATOMS = 10000
F = 128
R = 64
N_PAIRS = 320000
NC = 2
NS = 16
NW = NC * NS
PAIRS_PER_TILE = N_PAIRS // NW
CHUNK = 200
SUB = 40
NSUB = CHUNK // SUB
NBUF = 3
N_PAD = 10240
ROWS_PER_TILE = N_PAD // NS
ZH = 80


def _sc_scatter_body(iters, radial_hbm, idx0_hbm, idx1_hbm, out_hbm,
                     idx_v, rows_v, sems, sems_s, acc):
    c = lax.axis_index("c")
    s = lax.axis_index("s")
    wid = c * NS + s

    z16 = jnp.zeros((16,), jnp.float32)
    for i in range(ZH):
        for k in range(R // 16):
            rows_v[0, i, pl.ds(k * 16, 16)] = z16
    row0 = s * ROWS_PER_TILE
    for j in range(ROWS_PER_TILE // ZH):
        pltpu.sync_copy(rows_v.at[0, pl.ds(0, ZH)], acc.at[pl.ds(row0 + j * ZH, ZH)])
    plsc.subcore_barrier()

    pair0 = wid * (iters * CHUNK)

    def issue_load(j, b):
        pb = pair0 + j * CHUNK
        pltpu.async_copy(idx0_hbm.at[pl.ds(pb, CHUNK)], idx_v.at[b, 0], sems.at[b])
        pltpu.async_copy(idx1_hbm.at[pl.ds(pb, CHUNK)], idx_v.at[b, 1], sems.at[b])
        pltpu.async_copy(radial_hbm.at[pl.ds(pb, CHUNK)], rows_v.at[b], sems.at[b])

    def wait_load(j, b):
        pb = pair0 + j * CHUNK
        pltpu.make_async_copy(idx0_hbm.at[pl.ds(pb, CHUNK)], idx_v.at[b, 0], sems.at[b]).wait()
        pltpu.make_async_copy(idx1_hbm.at[pl.ds(pb, CHUNK)], idx_v.at[b, 1], sems.at[b]).wait()
        pltpu.make_async_copy(radial_hbm.at[pl.ds(pb, CHUNK)], rows_v.at[b], sems.at[b]).wait()

    for b in range(NBUF):
        issue_load(b, b)

    def scatter_refs(b, k):
        src = rows_v.at[b, pl.ds(k * SUB, SUB)]
        d0 = acc.at[idx_v.at[b, 0, pl.ds(k * SUB, SUB)]]
        d1 = acc.at[idx_v.at[b, 1, pl.ds(k * SUB, SUB)]]
        return src, d0, d1

    @pl.loop(0, iters, step=NBUF)
    def _pipeline(g):
        for b in range(NBUF):
            j = g + b

            @pl.when(j < iters)
            def _():
                wait_load(j, b)
                for k in range(NSUB):
                    src, d0, d1 = scatter_refs(b, k)
                    pltpu.sync_copy(src, d0, add=True)
                    pltpu.sync_copy(src, d1, add=True)
                nj = j + NBUF

                @pl.when(nj < iters)
                def _():
                    issue_load(nj, b)

    plsc.subcore_barrier()

    for j in range(ROWS_PER_TILE // ZH):
        pltpu.sync_copy(acc.at[pl.ds(row0 + j * ZH, ZH)], rows_v.at[0, pl.ds(0, ZH)])
        pltpu.sync_copy(rows_v.at[0, pl.ds(0, ZH)], out_hbm.at[c, pl.ds(row0 + j * ZH, ZH)])


@jax.jit
def _sc_scatter(radial_slice, idx0, idx1):
    iters = radial_slice.shape[0] // (NW * CHUNK)
    mesh = plsc.VectorSubcoreMesh(core_axis_name="c", subcore_axis_name="s")
    return pl.kernel(
        functools.partial(_sc_scatter_body, iters),
        out_type=jax.ShapeDtypeStruct((NC, N_PAD, R), jnp.float32),
        mesh=mesh,
        compiler_params=pltpu.CompilerParams(use_tc_tiling_on_sc=False),
        scratch_types=[
            pltpu.VMEM((NBUF, 2, CHUNK), jnp.int32),
            pltpu.VMEM((NBUF, CHUNK, R), jnp.float32),
            pltpu.SemaphoreType.DMA((NBUF,)),
            pltpu.SemaphoreType.DMA((NBUF,)),
            pltpu.VMEM_SHARED((N_PAD, R), jnp.float32),
        ],
    )(radial_slice, idx0, idx1)


_LOG2 = 0.6931471805599453


def _sp(x):
    return jnp.maximum(x, 0.0) + jnp.log(1.0 + jnp.exp(-jnp.abs(x))) - _LOG2


def _mm(x, w):
    return lax.dot_general(x, w, (((1,), (1,)), ((), ())),
                           preferred_element_type=jnp.float32)


def _dense_body(nslice, species_ref, feat_ref, *rest):
    part_refs = rest[:nslice]
    wrefs = rest[nslice:-2]
    energy_ref, outfeat_ref = rest[-2:]
    w = [r[...] for r in wrefs]
    (WI, bI, WJ, bJ, Wg,
     i10, i11, i12, i13, i20, i21, i22, i23, i30, i31, i32, i33,
     Wint, bint, gate,
     a10, a11, a12, a13, a20, a21, a22, a23,
     o10, o11, o12, o13,
     Wout, bout) = w

    def res_block(x, W1, b1, W2, b2):
        out = _mm(_sp(x), W1) + b1
        return _mm(_sp(out), W2) + b2 + x

    x = feat_ref[...]
    mask = species_ref[...] != -1
    af = _sp(x)
    g = _sp(_mm(af, WJ) + bJ)
    protoI = _sp(_mm(af, WI) + bI)
    A = part_refs[0][0] + part_refs[0][1]
    for pref in part_refs[1:]:
        A = A + pref[0] + pref[1]
    S = _mm(A, Wg)
    proto = S * g + jnp.where(mask, protoI, 0.0)
    msg = res_block(proto, i10, i11, i12, i13)
    msg = res_block(msg, i20, i21, i22, i23)
    msg = res_block(msg, i30, i31, i32, i33)
    dense = x * gate + _mm(_sp(msg), Wint) + bint
    dense = res_block(dense, a10, a11, a12, a13)
    dense = res_block(dense, a20, a21, a22, a23)
    t = res_block(dense, o10, o11, o12, o13)
    e = jnp.sum(_sp(t) * Wout + bout, axis=1, keepdims=True)
    energy_ref[...] = jnp.where(mask, e, 0.0)
    outfeat_ref[...] = jnp.where(mask, dense, 0.0)


def _dense_chain(species_flat, features, partials, weights, block_rows=2000):
    grid = (N_ATOMS // block_rows,)
    row_spec = lambda cols: pl.BlockSpec((block_rows, cols), lambda i: (i, 0))
    in_specs = [
        row_spec(1),
        row_spec(F),
    ] + [pl.BlockSpec((NC, block_rows, R), lambda i: (0, i, 0)) for _ in partials
    ] + [pl.BlockSpec(wi.shape, lambda i: (0, 0)) for wi in weights]
    out_specs = [row_spec(1), row_spec(F)]
    energies, out_features = pl.pallas_call(
        functools.partial(_dense_body, len(partials)),
        grid=grid,
        in_specs=in_specs,
        out_specs=out_specs,
        out_shape=[
            jax.ShapeDtypeStruct((N_ATOMS, 1), jnp.float32),
            jax.ShapeDtypeStruct((N_ATOMS, F), jnp.float32),
        ],
    )(species_flat, features, *partials, *weights)
    return energies, out_features


NSLICE = 1


def kernel(species, features, radial_aev, atom_index12, params):
    idx = atom_index12.astype(jnp.int32)
    ps = N_PAIRS // NSLICE
    partials = []
    for i in range(NSLICE):
        i0 = idx[0, i * ps:(i + 1) * ps]
        i1 = idx[1, i * ps:(i + 1) * ps]
        if partials:
            gate = (partials[-1][0, 0, 0] * 0.0).astype(jnp.int32)
            i0 = i0 + gate
        partials.append(_sc_scatter(radial_aev[i * ps:(i + 1) * ps], i0, i1))

    def lin2(p):
        W, b = p
        return [W, b.reshape(1, F)]

    pr = params
    weights = (
        lin2(pr['linearI']) + lin2(pr['linearJ']) + [pr['gating_linear_W']]
        + [t for blk in pr['inter_res'] for p in blk for t in lin2(p)]
        + lin2(pr['interaction_linear'])
        + [pr['gating_vector'].reshape(1, F)]
        + [t for blk in pr['atomic_res'] for p in blk for t in lin2(p)]
        + [t for blk in pr['output_res'] for p in blk for t in lin2(p)]
        + [pr['output_linear'][0],
           jnp.broadcast_to(pr['output_linear'][1].reshape(1, 1) / F, (1, F))]
    )
    species_flat = species.reshape(-1, 1).astype(jnp.int32)
    energies, out_features = _dense_chain(species_flat, features, partials, weights)
    return energies.reshape(species.shape[0], species.shape[1]), out_features


# speedup vs baseline: 1.0344x; 1.0329x over previous
import functools

import jax
import jax.numpy as jnp
from jax import lax
from jax.experimental import pallas as pl
from jax.experimental.pallas import tpu as pltpu
from jax.experimental.pallas import tpu_sc as plsc

N_ATOMS = 10000
F = 128
R = 64
N_PAIRS = 320000
NC = 2
NS = 16
NW = NC * NS
PAIRS_PER_TILE = N_PAIRS // NW
CHUNK = 200
SUB = 40
NSUB = CHUNK // SUB
NBUF = 3
N_PAD = 10240
ROWS_PER_TILE = N_PAD // NS
ZH = 80


def _sc_scatter_body(iters, radial_hbm, idx0_hbm, idx1_hbm, out_hbm,
                     idx_v, rows_v, sems, sems_s, acc):
    c = lax.axis_index("c")
    s = lax.axis_index("s")
    wid = c * NS + s

    z16 = jnp.zeros((16,), jnp.float32)
    for i in range(ZH):
        for k in range(R // 16):
            rows_v[0, i, pl.ds(k * 16, 16)] = z16
    row0 = s * ROWS_PER_TILE
    for j in range(ROWS_PER_TILE // ZH):
        pltpu.sync_copy(rows_v.at[0, pl.ds(0, ZH)], acc.at[pl.ds(row0 + j * ZH, ZH)])
    plsc.subcore_barrier()

    pair0 = wid * (iters * CHUNK)

    def issue_load(j, b):
        pb = pair0 + j * CHUNK
        pltpu.async_copy(idx0_hbm.at[pl.ds(pb, CHUNK)], idx_v.at[b, 0], sems.at[b])
        pltpu.async_copy(idx1_hbm.at[pl.ds(pb, CHUNK)], idx_v.at[b, 1], sems.at[b])
        pltpu.async_copy(radial_hbm.at[pl.ds(pb, CHUNK)], rows_v.at[b], sems.at[b])

    def wait_load(j, b):
        pb = pair0 + j * CHUNK
        pltpu.make_async_copy(idx0_hbm.at[pl.ds(pb, CHUNK)], idx_v.at[b, 0], sems.at[b]).wait()
        pltpu.make_async_copy(idx1_hbm.at[pl.ds(pb, CHUNK)], idx_v.at[b, 1], sems.at[b]).wait()
        pltpu.make_async_copy(radial_hbm.at[pl.ds(pb, CHUNK)], rows_v.at[b], sems.at[b]).wait()

    for b in range(NBUF):
        issue_load(b, b)

    def scatter_refs(b, k):
        src = rows_v.at[b, pl.ds(k * SUB, SUB)]
        d0 = acc.at[idx_v.at[b, 0, pl.ds(k * SUB, SUB)]]
        d1 = acc.at[idx_v.at[b, 1, pl.ds(k * SUB, SUB)]]
        return src, d0, d1

    @pl.loop(0, iters, step=NBUF)
    def _pipeline(g):
        for b in range(NBUF):
            j = g + b

            @pl.when(j < iters)
            def _():
                wait_load(j, b)
                descs = []
                for k in range(NSUB):
                    src, d0, d1 = scatter_refs(b, k)
                    descs.append(pltpu.async_copy(src, d0, sems_s.at[b], add=True))
                    descs.append(pltpu.async_copy(src, d1, sems_s.at[b], add=True))
                for d in descs:
                    d.wait()
                nj = j + NBUF

                @pl.when(nj < iters)
                def _():
                    issue_load(nj, b)

    plsc.subcore_barrier()

    for j in range(ROWS_PER_TILE // ZH):
        pltpu.sync_copy(acc.at[pl.ds(row0 + j * ZH, ZH)], rows_v.at[0, pl.ds(0, ZH)])
        pltpu.sync_copy(rows_v.at[0, pl.ds(0, ZH)], out_hbm.at[c, pl.ds(row0 + j * ZH, ZH)])


@jax.jit
def _sc_scatter(radial_slice, idx0, idx1):
    iters = radial_slice.shape[0] // (NW * CHUNK)
    mesh = plsc.VectorSubcoreMesh(core_axis_name="c", subcore_axis_name="s")
    return pl.kernel(
        functools.partial(_sc_scatter_body, iters),
        out_type=jax.ShapeDtypeStruct((NC, N_PAD, R), jnp.float32),
        mesh=mesh,
        compiler_params=pltpu.CompilerParams(use_tc_tiling_on_sc=False),
        scratch_types=[
            pltpu.VMEM((NBUF, 2, CHUNK), jnp.int32),
            pltpu.VMEM((NBUF, CHUNK, R), jnp.float32),
            pltpu.SemaphoreType.DMA((NBUF,)),
            pltpu.SemaphoreType.DMA((NBUF,)),
            pltpu.VMEM_SHARED((N_PAD, R), jnp.float32),
        ],
    )(radial_slice, idx0, idx1)


_LOG2 = 0.6931471805599453


def _sp(x):
    return jnp.maximum(x, 0.0) + jnp.log(1.0 + jnp.exp(-jnp.abs(x))) - _LOG2


def _mm(x, w):
    return lax.dot_general(x, w, (((1,), (1,)), ((), ())),
                           preferred_element_type=jnp.float32)


def _dense_body(nslice, species_ref, feat_ref, *rest):
    part_refs = rest[:nslice]
    wrefs = rest[nslice:-2]
    energy_ref, outfeat_ref = rest[-2:]
    w = [r[...] for r in wrefs]
    (WI, bI, WJ, bJ, Wg,
     i10, i11, i12, i13, i20, i21, i22, i23, i30, i31, i32, i33,
     Wint, bint, gate,
     a10, a11, a12, a13, a20, a21, a22, a23,
     o10, o11, o12, o13,
     Wout, bout) = w

    def res_block(x, W1, b1, W2, b2):
        out = _mm(_sp(x), W1) + b1
        return _mm(_sp(out), W2) + b2 + x

    x = feat_ref[...]
    mask = species_ref[...] != -1
    af = _sp(x)
    g = _sp(_mm(af, WJ) + bJ)
    protoI = _sp(_mm(af, WI) + bI)
    A = part_refs[0][0] + part_refs[0][1]
    for pref in part_refs[1:]:
        A = A + pref[0] + pref[1]
    S = _mm(A, Wg)
    proto = S * g + jnp.where(mask, protoI, 0.0)
    msg = res_block(proto, i10, i11, i12, i13)
    msg = res_block(msg, i20, i21, i22, i23)
    msg = res_block(msg, i30, i31, i32, i33)
    dense = x * gate + _mm(_sp(msg), Wint) + bint
    dense = res_block(dense, a10, a11, a12, a13)
    dense = res_block(dense, a20, a21, a22, a23)
    t = res_block(dense, o10, o11, o12, o13)
    e = jnp.sum(_sp(t) * Wout + bout, axis=1, keepdims=True)
    energy_ref[...] = jnp.where(mask, e, 0.0)
    outfeat_ref[...] = jnp.where(mask, dense, 0.0)


def _dense_chain(species_flat, features, partials, weights, block_rows=2000):
    grid = (N_ATOMS // block_rows,)
    row_spec = lambda cols: pl.BlockSpec((block_rows, cols), lambda i: (i, 0))
    in_specs = [
        row_spec(1),
        row_spec(F),
    ] + [pl.BlockSpec((NC, block_rows, R), lambda i: (0, i, 0)) for _ in partials
    ] + [pl.BlockSpec(wi.shape, lambda i: (0, 0)) for wi in weights]
    out_specs = [row_spec(1), row_spec(F)]
    energies, out_features = pl.pallas_call(
        functools.partial(_dense_body, len(partials)),
        grid=grid,
        in_specs=in_specs,
        out_specs=out_specs,
        out_shape=[
            jax.ShapeDtypeStruct((N_ATOMS, 1), jnp.float32),
            jax.ShapeDtypeStruct((N_ATOMS, F), jnp.float32),
        ],
    )(species_flat, features, *partials, *weights)
    return energies, out_features


NSLICE = 1


def kernel(species, features, radial_aev, atom_index12, params):
    idx = atom_index12.astype(jnp.int32)
    ps = N_PAIRS // NSLICE
    partials = []
    for i in range(NSLICE):
        i0 = idx[0, i * ps:(i + 1) * ps]
        i1 = idx[1, i * ps:(i + 1) * ps]
        if partials:
            gate = (partials[-1][0, 0, 0] * 0.0).astype(jnp.int32)
            i0 = i0 + gate
        partials.append(_sc_scatter(radial_aev[i * ps:(i + 1) * ps], i0, i1))

    def lin2(p):
        W, b = p
        return [W, b.reshape(1, F)]

    pr = params
    weights = (
        lin2(pr['linearI']) + lin2(pr['linearJ']) + [pr['gating_linear_W']]
        + [t for blk in pr['inter_res'] for p in blk for t in lin2(p)]
        + lin2(pr['interaction_linear'])
        + [pr['gating_vector'].reshape(1, F)]
        + [t for blk in pr['atomic_res'] for p in blk for t in lin2(p)]
        + [t for blk in pr['output_res'] for p in blk for t in lin2(p)]
        + [pr['output_linear'][0],
           jnp.broadcast_to(pr['output_linear'][1].reshape(1, 1) / F, (1, F))]
    )
    species_flat = species.reshape(-1, 1).astype(jnp.int32)
    energies, out_features = _dense_chain(species_flat, features, partials, weights)
    return energies.reshape(species.shape[0], species.shape[1]), out_features


# speedup vs baseline: 1.0404x; 1.0059x over previous
import functools

import jax
import jax.numpy as jnp
from jax import lax
from jax.experimental import pallas as pl
from jax.experimental.pallas import tpu as pltpu
from jax.experimental.pallas import tpu_sc as plsc

N_ATOMS = 10000
F = 128
R = 64
N_PAIRS = 320000
NC = 2
NS = 16
NW = NC * NS
PAIRS_PER_TILE = N_PAIRS // NW
CHUNK = 200
SUB = 40
NSUB = CHUNK // SUB
NBUF = 3
N_PAD = 10240
ROWS_PER_TILE = N_PAD // NS
ZH = 80


def _sc_scatter_body(iters, radial_hbm, idx0_hbm, idx1_hbm, out_hbm,
                     idx_v, rows_v, sems, sems_s, acc):
    c = lax.axis_index("c")
    s = lax.axis_index("s")
    wid = c * NS + s

    z16 = jnp.zeros((16,), jnp.float32)
    for i in range(ZH):
        for k in range(R // 16):
            rows_v[0, i, pl.ds(k * 16, 16)] = z16
    row0 = s * ROWS_PER_TILE
    zd = [pltpu.async_copy(rows_v.at[0, pl.ds(0, ZH)],
                           acc.at[pl.ds(row0 + j * ZH, ZH)], sems.at[0])
          for j in range(ROWS_PER_TILE // ZH)]
    for d in zd:
        d.wait()
    plsc.subcore_barrier()

    pair0 = wid * (iters * CHUNK)

    def issue_load(j, b):
        pb = pair0 + j * CHUNK
        pltpu.async_copy(idx0_hbm.at[pl.ds(pb, CHUNK)], idx_v.at[b, 0], sems.at[b])
        pltpu.async_copy(idx1_hbm.at[pl.ds(pb, CHUNK)], idx_v.at[b, 1], sems.at[b])
        pltpu.async_copy(radial_hbm.at[pl.ds(pb, CHUNK)], rows_v.at[b], sems.at[b])

    def wait_load(j, b):
        pb = pair0 + j * CHUNK
        pltpu.make_async_copy(idx0_hbm.at[pl.ds(pb, CHUNK)], idx_v.at[b, 0], sems.at[b]).wait()
        pltpu.make_async_copy(idx1_hbm.at[pl.ds(pb, CHUNK)], idx_v.at[b, 1], sems.at[b]).wait()
        pltpu.make_async_copy(radial_hbm.at[pl.ds(pb, CHUNK)], rows_v.at[b], sems.at[b]).wait()

    for b in range(NBUF):
        issue_load(b, b)

    def scatter_refs(b, k):
        src = rows_v.at[b, pl.ds(k * SUB, SUB)]
        d0 = acc.at[idx_v.at[b, 0, pl.ds(k * SUB, SUB)]]
        d1 = acc.at[idx_v.at[b, 1, pl.ds(k * SUB, SUB)]]
        return src, d0, d1

    @pl.loop(0, iters, step=NBUF)
    def _pipeline(g):
        for b in range(NBUF):
            j = g + b

            @pl.when(j < iters)
            def _():
                wait_load(j, b)
                descs = []
                for k in range(NSUB):
                    src, d0, d1 = scatter_refs(b, k)
                    descs.append(pltpu.async_copy(src, d0, sems_s.at[b], add=True))
                    descs.append(pltpu.async_copy(src, d1, sems_s.at[b], add=True))
                for d in descs:
                    d.wait()
                nj = j + NBUF

                @pl.when(nj < iters)
                def _():
                    issue_load(nj, b)

    plsc.subcore_barrier()

    wr = [None] * NBUF
    for j in range(ROWS_PER_TILE // ZH):
        b = j % NBUF
        if wr[b] is not None:
            wr[b].wait()
        pltpu.sync_copy(acc.at[pl.ds(row0 + j * ZH, ZH)], rows_v.at[b, pl.ds(0, ZH)])
        wr[b] = pltpu.async_copy(rows_v.at[b, pl.ds(0, ZH)],
                                 out_hbm.at[c, pl.ds(row0 + j * ZH, ZH)], sems.at[b])
    for d in wr:
        if d is not None:
            d.wait()


@jax.jit
def _sc_scatter(radial_slice, idx0, idx1):
    iters = radial_slice.shape[0] // (NW * CHUNK)
    mesh = plsc.VectorSubcoreMesh(core_axis_name="c", subcore_axis_name="s")
    return pl.kernel(
        functools.partial(_sc_scatter_body, iters),
        out_type=jax.ShapeDtypeStruct((NC, N_PAD, R), jnp.float32),
        mesh=mesh,
        compiler_params=pltpu.CompilerParams(use_tc_tiling_on_sc=False),
        scratch_types=[
            pltpu.VMEM((NBUF, 2, CHUNK), jnp.int32),
            pltpu.VMEM((NBUF, CHUNK, R), jnp.float32),
            pltpu.SemaphoreType.DMA((NBUF,)),
            pltpu.SemaphoreType.DMA((NBUF,)),
            pltpu.VMEM_SHARED((N_PAD, R), jnp.float32),
        ],
    )(radial_slice, idx0, idx1)


_LOG2 = 0.6931471805599453


def _sp(x):
    return jnp.maximum(x, 0.0) + jnp.log(1.0 + jnp.exp(-jnp.abs(x))) - _LOG2


def _mm(x, w):
    return lax.dot_general(x, w, (((1,), (1,)), ((), ())),
                           preferred_element_type=jnp.float32)


def _dense_body(nslice, species_ref, feat_ref, *rest):
    part_refs = rest[:nslice]
    wrefs = rest[nslice:-2]
    energy_ref, outfeat_ref = rest[-2:]
    w = [r[...] for r in wrefs]
    (WI, bI, WJ, bJ, Wg,
     i10, i11, i12, i13, i20, i21, i22, i23, i30, i31, i32, i33,
     Wint, bint, gate,
     a10, a11, a12, a13, a20, a21, a22, a23,
     o10, o11, o12, o13,
     Wout, bout) = w

    def res_block(x, W1, b1, W2, b2):
        out = _mm(_sp(x), W1) + b1
        return _mm(_sp(out), W2) + b2 + x

    x = feat_ref[...]
    mask = species_ref[...] != -1
    af = _sp(x)
    g = _sp(_mm(af, WJ) + bJ)
    protoI = _sp(_mm(af, WI) + bI)
    A = part_refs[0][0] + part_refs[0][1]
    for pref in part_refs[1:]:
        A = A + pref[0] + pref[1]
    S = _mm(A, Wg)
    proto = S * g + jnp.where(mask, protoI, 0.0)
    msg = res_block(proto, i10, i11, i12, i13)
    msg = res_block(msg, i20, i21, i22, i23)
    msg = res_block(msg, i30, i31, i32, i33)
    dense = x * gate + _mm(_sp(msg), Wint) + bint
    dense = res_block(dense, a10, a11, a12, a13)
    dense = res_block(dense, a20, a21, a22, a23)
    t = res_block(dense, o10, o11, o12, o13)
    e = jnp.sum(_sp(t) * Wout + bout, axis=1, keepdims=True)
    energy_ref[...] = jnp.where(mask, e, 0.0)
    outfeat_ref[...] = jnp.where(mask, dense, 0.0)


def _dense_chain(species_flat, features, partials, weights, block_rows=2000):
    grid = (N_ATOMS // block_rows,)
    row_spec = lambda cols: pl.BlockSpec((block_rows, cols), lambda i: (i, 0))
    in_specs = [
        row_spec(1),
        row_spec(F),
    ] + [pl.BlockSpec((NC, block_rows, R), lambda i: (0, i, 0)) for _ in partials
    ] + [pl.BlockSpec(wi.shape, lambda i: (0, 0)) for wi in weights]
    out_specs = [row_spec(1), row_spec(F)]
    energies, out_features = pl.pallas_call(
        functools.partial(_dense_body, len(partials)),
        grid=grid,
        in_specs=in_specs,
        out_specs=out_specs,
        out_shape=[
            jax.ShapeDtypeStruct((N_ATOMS, 1), jnp.float32),
            jax.ShapeDtypeStruct((N_ATOMS, F), jnp.float32),
        ],
    )(species_flat, features, *partials, *weights)
    return energies, out_features


NSLICE = 1


def kernel(species, features, radial_aev, atom_index12, params):
    idx = atom_index12.astype(jnp.int32)
    ps = N_PAIRS // NSLICE
    partials = []
    for i in range(NSLICE):
        i0 = idx[0, i * ps:(i + 1) * ps]
        i1 = idx[1, i * ps:(i + 1) * ps]
        if partials:
            gate = (partials[-1][0, 0, 0] * 0.0).astype(jnp.int32)
            i0 = i0 + gate
        partials.append(_sc_scatter(radial_aev[i * ps:(i + 1) * ps], i0, i1))

    def lin2(p):
        W, b = p
        return [W, b.reshape(1, F)]

    pr = params
    weights = (
        lin2(pr['linearI']) + lin2(pr['linearJ']) + [pr['gating_linear_W']]
        + [t for blk in pr['inter_res'] for p in blk for t in lin2(p)]
        + lin2(pr['interaction_linear'])
        + [pr['gating_vector'].reshape(1, F)]
        + [t for blk in pr['atomic_res'] for p in blk for t in lin2(p)]
        + [t for blk in pr['output_res'] for p in blk for t in lin2(p)]
        + [pr['output_linear'][0],
           jnp.broadcast_to(pr['output_linear'][1].reshape(1, 1) / F, (1, F))]
    )
    species_flat = species.reshape(-1, 1).astype(jnp.int32)
    energies, out_features = _dense_chain(species_flat, features, partials, weights)
    return energies.reshape(species.shape[0], species.shape[1]), out_features


# speedup vs baseline: 1.0407x; 1.0003x over previous
import functools

import jax
import jax.numpy as jnp
from jax import lax
from jax.experimental import pallas as pl
from jax.experimental.pallas import tpu as pltpu
from jax.experimental.pallas import tpu_sc as plsc

N_ATOMS = 10000
F = 128
R = 64
N_PAIRS = 320000
NC = 2
NS = 16
NW = NC * NS
PAIRS_PER_TILE = N_PAIRS // NW
CHUNK = 200
SUB = 40
NSUB = CHUNK // SUB
NBUF = 3
N_PAD = 10240
ROWS_PER_TILE = N_PAD // NS
ZH = 80


def _sc_scatter_body(iters, radial_hbm, idx0_hbm, idx1_hbm, out_hbm,
                     idx_v, rows_v, sems, sems_s, acc):
    c = lax.axis_index("c")
    s = lax.axis_index("s")
    wid = c * NS + s

    z16 = jnp.zeros((16,), jnp.float32)
    for i in range(ZH):
        for k in range(R // 16):
            rows_v[0, i, pl.ds(k * 16, 16)] = z16
    row0 = s * ROWS_PER_TILE
    zd = [pltpu.async_copy(rows_v.at[0, pl.ds(0, ZH)],
                           acc.at[pl.ds(row0 + j * ZH, ZH)], sems.at[0])
          for j in range(ROWS_PER_TILE // ZH)]
    for d in zd:
        d.wait()
    plsc.subcore_barrier()

    pair0 = wid * (iters * CHUNK)

    def issue_load(j, b):
        pb = pair0 + j * CHUNK
        pltpu.async_copy(idx0_hbm.at[pl.ds(pb, CHUNK)], idx_v.at[b, 0], sems.at[b])
        pltpu.async_copy(idx1_hbm.at[pl.ds(pb, CHUNK)], idx_v.at[b, 1], sems.at[b])
        pltpu.async_copy(radial_hbm.at[pl.ds(pb, CHUNK)], rows_v.at[b], sems.at[b])

    def wait_load(j, b):
        pb = pair0 + j * CHUNK
        pltpu.make_async_copy(idx0_hbm.at[pl.ds(pb, CHUNK)], idx_v.at[b, 0], sems.at[b]).wait()
        pltpu.make_async_copy(idx1_hbm.at[pl.ds(pb, CHUNK)], idx_v.at[b, 1], sems.at[b]).wait()
        pltpu.make_async_copy(radial_hbm.at[pl.ds(pb, CHUNK)], rows_v.at[b], sems.at[b]).wait()

    for b in range(NBUF):
        issue_load(b, b)

    def scatter_refs(b, k):
        src = rows_v.at[b, pl.ds(k * SUB, SUB)]
        d0 = acc.at[idx_v.at[b, 0, pl.ds(k * SUB, SUB)]]
        d1 = acc.at[idx_v.at[b, 1, pl.ds(k * SUB, SUB)]]
        return src, d0, d1

    @pl.loop(0, iters, step=NBUF)
    def _pipeline(g):
        for b in range(NBUF):
            j = g + b

            @pl.when(j < iters)
            def _():
                wait_load(j, b)
                descs = []
                for k in range(NSUB):
                    src, d0, d1 = scatter_refs(b, k)
                    descs.append(pltpu.async_copy(src, d0, sems_s.at[b], add=True))
                    descs.append(pltpu.async_copy(src, d1, sems_s.at[b], add=True))
                for d in descs:
                    d.wait()
                nj = j + NBUF

                @pl.when(nj < iters)
                def _():
                    issue_load(nj, b)

    plsc.subcore_barrier()

    wr = [None] * NBUF
    for j in range(ROWS_PER_TILE // ZH):
        b = j % NBUF
        if wr[b] is not None:
            wr[b].wait()
        pltpu.sync_copy(acc.at[pl.ds(row0 + j * ZH, ZH)], rows_v.at[b, pl.ds(0, ZH)])
        wr[b] = pltpu.async_copy(rows_v.at[b, pl.ds(0, ZH)],
                                 out_hbm.at[c, pl.ds(row0 + j * ZH, ZH)], sems.at[b])
    for d in wr:
        if d is not None:
            d.wait()


@jax.jit
def _sc_scatter(radial_slice, idx0, idx1):
    iters = radial_slice.shape[0] // (NW * CHUNK)
    mesh = plsc.VectorSubcoreMesh(core_axis_name="c", subcore_axis_name="s")
    return pl.kernel(
        functools.partial(_sc_scatter_body, iters),
        out_type=jax.ShapeDtypeStruct((NC, N_PAD, R), jnp.float32),
        mesh=mesh,
        compiler_params=pltpu.CompilerParams(use_tc_tiling_on_sc=False),
        scratch_types=[
            pltpu.VMEM((NBUF, 2, CHUNK), jnp.int32),
            pltpu.VMEM((NBUF, CHUNK, R), jnp.float32),
            pltpu.SemaphoreType.DMA((NBUF,)),
            pltpu.SemaphoreType.DMA((NBUF,)),
            pltpu.VMEM_SHARED((N_PAD, R), jnp.float32),
        ],
    )(radial_slice, idx0, idx1)


_LOG2 = 0.6931471805599453


def _sp(x):
    return jnp.maximum(x, 0.0) + jnp.log(1.0 + jnp.exp(-jnp.abs(x))) - _LOG2


def _mm(x, w):
    return lax.dot_general(x, w, (((1,), (1,)), ((), ())),
                           preferred_element_type=jnp.float32)


def _dense_body(nslice, species_ref, feat_ref, *rest):
    part_refs = rest[:nslice]
    wrefs = rest[nslice:-2]
    energy_ref, outfeat_ref = rest[-2:]
    w = [r[...] for r in wrefs]
    (WI, bI, WJ, bJ, Wg,
     i10, i11, i12, i13, i20, i21, i22, i23, i30, i31, i32, i33,
     Wint, bint, gate,
     a10, a11, a12, a13, a20, a21, a22, a23,
     o10, o11, o12, o13,
     Wout, bout) = w

    def res_block(x, W1, b1, W2, b2):
        out = _mm(_sp(x), W1) + b1
        return _mm(_sp(out), W2) + b2 + x

    x = feat_ref[...]
    mask = species_ref[...] != -1
    af = _sp(x)
    g = _sp(_mm(af, WJ) + bJ)
    protoI = _sp(_mm(af, WI) + bI)
    A = part_refs[0][0] + part_refs[0][1]
    for pref in part_refs[1:]:
        A = A + pref[0] + pref[1]
    S = _mm(A, Wg)
    proto = S * g + jnp.where(mask, protoI, 0.0)
    msg = res_block(proto, i10, i11, i12, i13)
    msg = res_block(msg, i20, i21, i22, i23)
    msg = res_block(msg, i30, i31, i32, i33)
    dense = x * gate + _mm(_sp(msg), Wint) + bint
    dense = res_block(dense, a10, a11, a12, a13)
    dense = res_block(dense, a20, a21, a22, a23)
    t = res_block(dense, o10, o11, o12, o13)
    e = jnp.sum(_sp(t) * Wout + bout, axis=1, keepdims=True)
    energy_ref[...] = jnp.where(mask, e, 0.0)
    outfeat_ref[...] = jnp.where(mask, dense, 0.0)


def _dense_chain(species_flat, features, partials, weights, block_rows=2000):
    grid = (N_ATOMS // block_rows,)
    row_spec = lambda cols: pl.BlockSpec((block_rows, cols), lambda i: (i, 0))
    in_specs = [
        row_spec(1),
        row_spec(F),
    ] + [pl.BlockSpec((NC, block_rows, R), lambda i: (0, i, 0)) for _ in partials
    ] + [pl.BlockSpec(wi.shape, lambda i: (0, 0)) for wi in weights]
    out_specs = [row_spec(1), row_spec(F)]
    energies, out_features = pl.pallas_call(
        functools.partial(_dense_body, len(partials)),
        grid=grid,
        in_specs=in_specs,
        out_specs=out_specs,
        out_shape=[
            jax.ShapeDtypeStruct((N_ATOMS, 1), jnp.float32),
            jax.ShapeDtypeStruct((N_ATOMS, F), jnp.float32),
        ],
    )(species_flat, features, *partials, *weights)
    return energies, out_features


def kernel(species, features, radial_aev, atom_index12, params):
    idx = atom_index12.astype(jnp.int32)
    partials = [_sc_scatter(radial_aev, idx[0], idx[1])]

    def lin2(p):
        W, b = p
        return [W, b.reshape(1, F)]

    pr = params
    weights = (
        lin2(pr['linearI']) + lin2(pr['linearJ']) + [pr['gating_linear_W']]
        + [t for blk in pr['inter_res'] for p in blk for t in lin2(p)]
        + lin2(pr['interaction_linear'])
        + [pr['gating_vector'].reshape(1, F)]
        + [t for blk in pr['atomic_res'] for p in blk for t in lin2(p)]
        + [t for blk in pr['output_res'] for p in blk for t in lin2(p)]
        + [pr['output_linear'][0],
           jnp.broadcast_to(pr['output_linear'][1].reshape(1, 1) / F, (1, F))]
    )
    species_flat = species.reshape(-1, 1).astype(jnp.int32)
    energies, out_features = _dense_chain(species_flat, features, partials, weights)
    return energies.reshape(species.shape[0], species.shape[1]), out_features
